# baseline (speedup 1.0000x reference)
import functools

import jax
import jax.numpy as jnp
from jax import lax
from jax.experimental import pallas as pl
from jax.experimental.pallas import tpu as pltpu
from jax.experimental.pallas import tpu_sc as plsc

D = 64
SCALE = 8.0
NC, NS, L = 2, 16, 16
NW = NC * NS
NTOK = 4096
NSEQ = 200
VOC = 1000000
NBF = VOC // 128
CH = 128
NCH = NSEQ * (NTOK // CH)
CPW = NCH // NW
CBLK = NTOK // CH

_MESH = dict(core_axis_name="c", subcore_axis_name="s",
             num_cores=NC, num_subcores=NS)


def _transpose_table(tab_t, tail):
    mesh = plsc.VectorSubcoreMesh(**_MESH)

    @functools.partial(
        pl.kernel,
        out_type=jax.ShapeDtypeStruct((VOC // 2, 128), jnp.float32),
        mesh=mesh,
        compiler_params=pltpu.CompilerParams(
            use_tc_tiling_on_sc=True, needs_layout_passes=False
        ),
        scratch_types=[
            pltpu.VMEM((8, 8, 128), jnp.float32),
            pltpu.VMEM((8, 8, 128), jnp.float32),
            pltpu.VMEM((64, 130), jnp.float32),
            pltpu.VMEM((64, 130), jnp.float32),
            pltpu.SemaphoreType.DMA,
            pltpu.SemaphoreType.DMA,
            pltpu.SemaphoreType.DMA,
            pltpu.SemaphoreType.DMA,
        ],
    )
    def ka(tab_hbm, tail_hbm, out_hbm, in0, in1, st0, st1, g0, g1, s0, s1):
        wid = lax.axis_index("s") * NC + lax.axis_index("c")
        nbw = (NBF + 31 - wid) // 32
        lanes = lax.iota(jnp.int32, L)

        def start_load(v0, inb, sem):
            v0 = pl.multiple_of(v0, 128)
            for t in range(8):
                pltpu.async_copy(
                    tab_hbm.at[pl.ds(8 * t, 8), pl.ds(v0, 128)],
                    inb.at[t], sem)

        def wait_load(inb, sem):
            for t in range(8):
                pltpu.make_async_copy(tab_hbm.at[pl.ds(0, 8), pl.ds(0, 128)],
                                      inb.at[t], sem).wait()

        parity64 = (lanes % 2) * 64

        def permute(inb, stv):
            for t in range(8):
                for r in range(8):
                    dcol = parity64 + (8 * t + r)
                    for m in range(8):
                        vec = inb.at[t, r, pl.ds(L * m, L)][...]
                        plsc.store_scatter(stv, [(lanes + L * m) // 2, dcol],
                                           vec)

        def start_store(v0, stv, sem):
            w0 = pl.multiple_of(v0 // 2, 64)
            pltpu.async_copy(stv.at[pl.ds(0, 64), pl.ds(0, 128)],
                             out_hbm.at[pl.ds(w0, 64)], sem)

        def wait_store(stv, sem):
            pltpu.make_async_copy(stv.at[pl.ds(0, 64), pl.ds(0, 128)],
                                  out_hbm.at[pl.ds(0, 64)], sem).wait()

        start_load(wid * 128, in0, g0)

        @pl.loop(0, 246, step=2)
        def _(n):
            v0 = (wid + n * 32) * 128

            @pl.when(n + 1 < nbw)
            def _():
                start_load(v0 + 32 * 128, in1, g1)

            @pl.when(n < nbw)
            def _():
                wait_load(in0, g0)

                @pl.when(n > 0)
                def _():
                    wait_store(st0, s0)

                permute(in0, st0)
                start_store(v0, st0, s0)

            @pl.when(n + 2 < nbw)
            def _():
                start_load(v0 + 64 * 128, in0, g0)

            @pl.when(n + 1 < nbw)
            def _():
                wait_load(in1, g1)

                @pl.when(n > 0)
                def _():
                    wait_store(st1, s1)

                permute(in1, st1)
                start_store(v0 + 32 * 128, st1, s1)

        wait_store(st0, s0)

        @pl.when(nbw > 1)
        def _():
            wait_store(st1, s1)

        @pl.when(wid == 31)
        def _():
            pltpu.sync_copy(tail_hbm, out_hbm.at[pl.ds(NBF * 64, 32)])

    return ka(tab_t, tail)


def _gather_assemble(table, tok4):
    mesh = plsc.VectorSubcoreMesh(**_MESH)

    @functools.partial(
        pl.kernel,
        out_type=jax.ShapeDtypeStruct((NSEQ * 8 * CBLK, 8, 128), jnp.float32),
        mesh=mesh,
        compiler_params=pltpu.CompilerParams(
            use_tc_tiling_on_sc=False, needs_layout_passes=False
        ),
        scratch_types=[
            pltpu.VMEM((CH,), jnp.int32),
            pltpu.VMEM((CH,), jnp.int32),
            pltpu.VMEM((CH, D), jnp.float32),
            pltpu.VMEM((CH, D), jnp.float32),
            pltpu.VMEM((64, 129), jnp.float32),
            pltpu.VMEM((64, 129), jnp.float32),
            pltpu.SemaphoreType.DMA,
            pltpu.SemaphoreType.DMA,
            pltpu.SemaphoreType.DMA,
            pltpu.SemaphoreType.DMA,
        ],
    )
    def kb(tab_hbm, tok_hbm, out_hbm,
           idx0, idx1, rows0, rows1, st0, st1, g0, g1, s0, s1):
        wid = lax.axis_index("s") * NC + lax.axis_index("c")
        ch_base = wid * CPW
        lanes = lax.iota(jnp.int32, L)

        def load_idx(ch, idxv):
            j = ch // CBLK
            cb = ch % CBLK
            pltpu.sync_copy(tok_hbm.at[j // 8, cb, j % 8], idxv)

        def start_gather(idxv, rowsv, sem):
            pltpu.async_copy(tab_hbm.at[idxv], rowsv, sem)

        def wait_gather(rowsv, sem):
            pltpu.make_async_copy(tab_hbm.at[pl.ds(0, CH)], rowsv, sem).wait()

        def assemble(rowsv, stv):
            @plsc.parallel_loop(0, CH, unroll=8)
            def _(q):
                qv = jnp.broadcast_to(q, (L,))
                for kk in range(D // L):
                    vec = rowsv.at[q, pl.ds(L * kk, L)][...] * SCALE
                    plsc.store_scatter(stv, [lanes + L * kk, qv], vec)

        def start_stores(stv, ch, sem):
            j = ch // CBLK
            cb = ch % CBLK
            base = j * 8 * CBLK + cb
            for t in range(8):
                pltpu.async_copy(stv.at[pl.ds(8 * t, 8), pl.ds(0, 128)],
                                 out_hbm.at[base + CBLK * t], sem)

        def wait_stores(stv, sem):
            for t in range(8):
                pltpu.make_async_copy(stv.at[pl.ds(0, 8), pl.ds(0, 128)],
                                      out_hbm.at[0], sem).wait()

        load_idx(ch_base, idx0)
        start_gather(idx0, rows0, g0)

        @pl.loop(0, CPW, step=2)
        def _(kk):
            ch0 = ch_base + kk
            load_idx(ch0 + 1, idx1)
            start_gather(idx1, rows1, g1)
            wait_gather(rows0, g0)

            @pl.when(kk > 0)
            def _():
                wait_stores(st0, s0)

            assemble(rows0, st0)
            start_stores(st0, ch0, s0)

            @pl.when(kk + 2 < CPW)
            def _():
                load_idx(ch0 + 2, idx0)
                start_gather(idx0, rows0, g0)

            wait_gather(rows1, g1)

            @pl.when(kk > 0)
            def _():
                wait_stores(st1, s1)

            assemble(rows1, st1)
            start_stores(st1, ch0 + 1, s1)

        wait_stores(st0, s0)
        wait_stores(st1, s1)

    return kb(table, tok4)


def kernel(tokens, embedding):
    tab_wide = _transpose_table(embedding.T,
                                embedding[NBF * 128:].reshape(32, 128))
    tab_lin = tab_wide.reshape(VOC, D)
    tok4 = (tokens.astype(jnp.int32).T
            .reshape(NSEQ // 8, 8, NTOK // 128, 128)
            .transpose(0, 2, 1, 3))
    out3 = _gather_assemble(tab_lin, tok4)
    out5 = out3.reshape(NSEQ, 8, CBLK, 8, 128)
    return out5.transpose(2, 4, 0, 1, 3).reshape(NTOK, NSEQ, D)


# speedup vs baseline: 1.2356x; 1.2356x over previous
import functools

import jax
import jax.numpy as jnp
from jax import lax
from jax.experimental import pallas as pl
from jax.experimental.pallas import tpu as pltpu
from jax.experimental.pallas import tpu_sc as plsc

D = 64
SCALE = 8.0
NC, NS, L = 2, 16, 16
NW = NC * NS
NTOK = 4096
NSEQ = 200
VOC = 1000000
NBF = VOC // 128
CH = 128
NCH = NSEQ * (NTOK // CH)
CPW = NCH // NW
CBLK = NTOK // CH

_MESH = dict(core_axis_name="c", subcore_axis_name="s",
             num_cores=NC, num_subcores=NS)


def _transpose_table(tab_t, tail):
    mesh = plsc.VectorSubcoreMesh(**_MESH)

    @functools.partial(
        pl.kernel,
        out_type=jax.ShapeDtypeStruct((VOC // 2, 128), jnp.float32),
        mesh=mesh,
        compiler_params=pltpu.CompilerParams(
            use_tc_tiling_on_sc=True, needs_layout_passes=False
        ),
        scratch_types=[
            pltpu.VMEM((8, 8, 128), jnp.float32),
            pltpu.VMEM((8, 8, 128), jnp.float32),
            pltpu.VMEM((64, 130), jnp.float32),
            pltpu.VMEM((64, 130), jnp.float32),
            pltpu.SemaphoreType.DMA,
            pltpu.SemaphoreType.DMA,
            pltpu.SemaphoreType.DMA,
            pltpu.SemaphoreType.DMA,
        ],
    )
    def ka(tab_hbm, tail_hbm, out_hbm, in0, in1, st0, st1, g0, g1, s0, s1):
        wid = lax.axis_index("s") * NC + lax.axis_index("c")
        nbw = (NBF + 31 - wid) // 32
        lanes = lax.iota(jnp.int32, L)

        def start_load(v0, inb, sem):
            v0 = pl.multiple_of(v0, 128)
            for t in range(8):
                pltpu.async_copy(
                    tab_hbm.at[pl.ds(8 * t, 8), pl.ds(v0, 128)],
                    inb.at[t], sem)

        def wait_load(inb, sem):
            for t in range(8):
                pltpu.make_async_copy(tab_hbm.at[pl.ds(0, 8), pl.ds(0, 128)],
                                      inb.at[t], sem).wait()

        parity64 = (lanes % 2) * 64
        rows_m = [(lanes + L * m) // 2 for m in range(8)]

        def permute(inb, stv):
            @plsc.parallel_loop(0, 8, unroll=2)
            def _(t):
                for r in range(8):
                    dcol = parity64 + (8 * t + r)
                    for m in range(8):
                        vec = inb.at[t, r, pl.ds(L * m, L)][...]
                        plsc.store_scatter(stv, [rows_m[m], dcol], vec)

        def start_store(v0, stv, sem):
            w0 = pl.multiple_of(v0 // 2, 64)
            pltpu.async_copy(stv.at[pl.ds(0, 64), pl.ds(0, 128)],
                             out_hbm.at[pl.ds(w0, 64)], sem)

        def wait_store(stv, sem):
            pltpu.make_async_copy(stv.at[pl.ds(0, 64), pl.ds(0, 128)],
                                  out_hbm.at[pl.ds(0, 64)], sem).wait()

        start_load(wid * 128, in0, g0)

        @pl.loop(0, 246, step=2)
        def _(n):
            v0 = (wid + n * 32) * 128

            @pl.when(n + 1 < nbw)
            def _():
                start_load(v0 + 32 * 128, in1, g1)

            @pl.when(n < nbw)
            def _():
                wait_load(in0, g0)

                @pl.when(n > 0)
                def _():
                    wait_store(st0, s0)

                permute(in0, st0)
                start_store(v0, st0, s0)

            @pl.when(n + 2 < nbw)
            def _():
                start_load(v0 + 64 * 128, in0, g0)

            @pl.when(n + 1 < nbw)
            def _():
                wait_load(in1, g1)

                @pl.when(n > 0)
                def _():
                    wait_store(st1, s1)

                permute(in1, st1)
                start_store(v0 + 32 * 128, st1, s1)

        wait_store(st0, s0)

        @pl.when(nbw > 1)
        def _():
            wait_store(st1, s1)

        @pl.when(wid == 31)
        def _():
            pltpu.sync_copy(tail_hbm, out_hbm.at[pl.ds(NBF * 64, 32)])

    return ka(tab_t, tail)


def _gather_assemble(table, tok4):
    mesh = plsc.VectorSubcoreMesh(**_MESH)

    @functools.partial(
        pl.kernel,
        out_type=jax.ShapeDtypeStruct((NSEQ * 8 * CBLK, 8, 128), jnp.float32),
        mesh=mesh,
        compiler_params=pltpu.CompilerParams(
            use_tc_tiling_on_sc=False, needs_layout_passes=False
        ),
        scratch_types=[
            pltpu.VMEM((CH,), jnp.int32),
            pltpu.VMEM((CH,), jnp.int32),
            pltpu.VMEM((CH, D), jnp.float32),
            pltpu.VMEM((CH, D), jnp.float32),
            pltpu.VMEM((64, 129), jnp.float32),
            pltpu.VMEM((64, 129), jnp.float32),
            pltpu.SemaphoreType.DMA,
            pltpu.SemaphoreType.DMA,
            pltpu.SemaphoreType.DMA,
            pltpu.SemaphoreType.DMA,
        ],
    )
    def kb(tab_hbm, tok_hbm, out_hbm,
           idx0, idx1, rows0, rows1, st0, st1, g0, g1, s0, s1):
        wid = lax.axis_index("s") * NC + lax.axis_index("c")
        ch_base = wid * CPW
        lanes = lax.iota(jnp.int32, L)

        def load_idx(ch, idxv):
            j = ch // CBLK
            cb = ch % CBLK
            pltpu.sync_copy(tok_hbm.at[j // 8, cb, j % 8], idxv)

        def start_gather(idxv, rowsv, sem):
            pltpu.async_copy(tab_hbm.at[idxv], rowsv, sem)

        def wait_gather(rowsv, sem):
            pltpu.make_async_copy(tab_hbm.at[pl.ds(0, CH)], rowsv, sem).wait()

        def assemble(rowsv, stv):
            @plsc.parallel_loop(0, CH, unroll=8)
            def _(q):
                qv = jnp.broadcast_to(q, (L,))
                for kk in range(D // L):
                    vec = rowsv.at[q, pl.ds(L * kk, L)][...] * SCALE
                    plsc.store_scatter(stv, [lanes + L * kk, qv], vec)

        def start_stores(stv, ch, sem):
            j = ch // CBLK
            cb = ch % CBLK
            base = j * 8 * CBLK + cb
            for t in range(8):
                pltpu.async_copy(stv.at[pl.ds(8 * t, 8), pl.ds(0, 128)],
                                 out_hbm.at[base + CBLK * t], sem)

        def wait_stores(stv, sem):
            for t in range(8):
                pltpu.make_async_copy(stv.at[pl.ds(0, 8), pl.ds(0, 128)],
                                      out_hbm.at[0], sem).wait()

        load_idx(ch_base, idx0)
        start_gather(idx0, rows0, g0)

        @pl.loop(0, CPW, step=2)
        def _(kk):
            ch0 = ch_base + kk
            load_idx(ch0 + 1, idx1)
            start_gather(idx1, rows1, g1)
            wait_gather(rows0, g0)

            @pl.when(kk > 0)
            def _():
                wait_stores(st0, s0)

            assemble(rows0, st0)
            start_stores(st0, ch0, s0)

            @pl.when(kk + 2 < CPW)
            def _():
                load_idx(ch0 + 2, idx0)
                start_gather(idx0, rows0, g0)

            wait_gather(rows1, g1)

            @pl.when(kk > 0)
            def _():
                wait_stores(st1, s1)

            assemble(rows1, st1)
            start_stores(st1, ch0 + 1, s1)

        wait_stores(st0, s0)
        wait_stores(st1, s1)

    return kb(table, tok4)


def kernel(tokens, embedding):
    tab_wide = _transpose_table(embedding.T,
                                embedding[NBF * 128:].reshape(32, 128))
    tab_lin = tab_wide.reshape(VOC, D)
    tok4 = (tokens.astype(jnp.int32).T
            .reshape(NSEQ // 8, 8, NTOK // 128, 128)
            .transpose(0, 2, 1, 3))
    out3 = _gather_assemble(tab_lin, tok4)
    out5 = out3.reshape(NSEQ, 8, CBLK, 8, 128)
    return out5.transpose(2, 4, 0, 1, 3).reshape(NTOK, NSEQ, D)


# speedup vs baseline: 1.8007x; 1.4573x over previous
import functools

import jax
import jax.numpy as jnp
from jax import lax
from jax.experimental import pallas as pl
from jax.experimental.pallas import tpu as pltpu
from jax.experimental.pallas import tpu_sc as plsc

D = 64
SCALE = 8.0
NC, NS, L = 2, 16, 16
NW = NC * NS
NTOK = 4096
NSEQ = 200
VOC = 1000000
CH = 128
NCH = NSEQ * (NTOK // CH)
CPW = NCH // NW
CBLK = NTOK // CH
OUT_FLAT = NSEQ * 8 * CBLK * 8 * 128


def _sc_embed(table128, tok4):
    mesh = plsc.VectorSubcoreMesh(
        core_axis_name="c", subcore_axis_name="s", num_cores=NC, num_subcores=NS
    )

    @functools.partial(
        pl.kernel,
        out_type=jax.ShapeDtypeStruct((OUT_FLAT // 1024, 8, 128), jnp.float32),
        mesh=mesh,
        compiler_params=pltpu.CompilerParams(
            use_tc_tiling_on_sc=False, needs_layout_passes=False
        ),
        scratch_types=[
            pltpu.VMEM((CH,), jnp.int32),
            pltpu.VMEM((CH,), jnp.int32),
            pltpu.VMEM((CH, 128), jnp.float32),
            pltpu.VMEM((CH, 128), jnp.float32),
            pltpu.VMEM((64, 129), jnp.float32),
            pltpu.VMEM((64, 129), jnp.float32),
            pltpu.SemaphoreType.DMA,
            pltpu.SemaphoreType.DMA,
            pltpu.SemaphoreType.DMA,
            pltpu.SemaphoreType.DMA,
        ],
    )
    def k(tab_hbm, tok_hbm, out_hbm,
          idx0, idx1, rows0, rows1, st0, st1, g0, g1, s0, s1):
        wid = lax.axis_index("s") * NC + lax.axis_index("c")
        ch_base = wid * CPW
        lanes = lax.iota(jnp.int32, L)

        def load_idx(ch, idxv):
            j = ch // CBLK
            cb = ch % CBLK
            pltpu.sync_copy(tok_hbm.at[j // 8, cb, j % 8], idxv)

        def start_gather(idxv, rowsv, sem):
            pltpu.async_copy(tab_hbm.at[idxv], rowsv, sem)

        def wait_gather(rowsv, sem):
            pltpu.make_async_copy(tab_hbm.at[pl.ds(0, CH)], rowsv, sem).wait()

        def assemble(rowsv, stv):
            @plsc.parallel_loop(0, CH, unroll=8)
            def _(q):
                qv = jnp.broadcast_to(q, (L,))
                for kk in range(D // L):
                    vec = rowsv.at[q, pl.ds(L * kk, L)][...] * SCALE
                    plsc.store_scatter(stv, [lanes + L * kk, qv], vec)

        def start_stores(stv, ch, sem):
            j = ch // CBLK
            cb = ch % CBLK
            base = j * 8 * CBLK + cb
            for t in range(8):
                pltpu.async_copy(stv.at[pl.ds(8 * t, 8), pl.ds(0, 128)],
                                 out_hbm.at[base + CBLK * t],
                                 sem)

        def wait_stores(stv, sem):
            for t in range(8):
                pltpu.make_async_copy(stv.at[pl.ds(0, 8), pl.ds(0, 128)],
                                      out_hbm.at[0],
                                      sem).wait()

        load_idx(ch_base, idx0)
        start_gather(idx0, rows0, g0)

        @pl.loop(0, CPW, step=2)
        def _(kk):
            ch0 = ch_base + kk
            load_idx(ch0 + 1, idx1)
            start_gather(idx1, rows1, g1)
            wait_gather(rows0, g0)

            @pl.when(kk > 0)
            def _():
                wait_stores(st0, s0)

            assemble(rows0, st0)
            start_stores(st0, ch0, s0)

            @pl.when(kk + 2 < CPW)
            def _():
                load_idx(ch0 + 2, idx0)
                start_gather(idx0, rows0, g0)

            wait_gather(rows1, g1)

            @pl.when(kk > 0)
            def _():
                wait_stores(st1, s1)

            assemble(rows1, st1)
            start_stores(st1, ch0 + 1, s1)

        wait_stores(st0, s0)
        wait_stores(st1, s1)

    return k(table128, tok4)


def kernel(tokens, embedding):
    table128 = jnp.pad(embedding, ((0, 0), (0, 128 - D)))
    tok4 = (tokens.astype(jnp.int32).T
            .reshape(NSEQ // 8, 8, NTOK // 128, 128)
            .transpose(0, 2, 1, 3))
    out3 = _sc_embed(table128, tok4)
    out5 = out3.reshape(NSEQ, 8, CBLK, 8, 128)
    return out5.transpose(2, 4, 0, 1, 3).reshape(NTOK, NSEQ, D)
